# baseline (speedup 1.0000x reference)
import functools

import jax
import jax.numpy as jnp
from jax import lax
from jax.experimental import pallas as pl
from jax.experimental.pallas import tpu as pltpu
from jax.experimental.pallas import tpu_sc as plsc

PROMPT_LEN = 16
D = 768
SUF = 110
CTX = 1 + PROMPT_LEN + SUF
CORE_W = PROMPT_LEN * D
SUF_W = SUF * D
ROW_W = CTX * D
ROWS = 512
NUM_CORES = 2
NUM_SUBCORES = 16
NW = NUM_CORES * NUM_SUBCORES
RPW = ROWS // NW
CH = 8
NCH = RPW // CH
SUF_REP = 4

_mesh = plsc.VectorSubcoreMesh(core_axis_name="c", subcore_axis_name="s")


@functools.partial(
    pl.kernel,
    out_type=(
        jax.ShapeDtypeStruct((ROWS, ROW_W), jnp.float32),
        jax.ShapeDtypeStruct((ROWS, CTX), jnp.int32),
    ),
    mesh=_mesh,
    scratch_types=[
        pltpu.VMEM((RPW,), jnp.int32),
        pltpu.VMEM((CH, CORE_W), jnp.float32),
        pltpu.VMEM((CH, D), jnp.float32),
        pltpu.VMEM((RPW, CTX), jnp.int32),
        pltpu.VMEM_SHARED((SUF_REP, SUF_W), jnp.float32),
        pltpu.SemaphoreType.DMA,
        pltpu.SemaphoreType.DMA,
        pltpu.SemaphoreType.DMA,
    ],
)
def _assemble(idx_hbm, prompt_hbm, pre_hbm, suf_hbm, tok_hbm,
              out_emb, out_tok,
              idx_v, core_v, pre_v, tok_v, suf_sh, gsem, osem, ssem):
    cid = lax.axis_index("c")
    sid = lax.axis_index("s")
    wid = sid * NUM_CORES + cid
    base = wid * RPW

    @pl.when(sid == 0)
    def _():
        for k in range(SUF_REP):
            pltpu.sync_copy(suf_hbm, suf_sh.at[pl.ds(k, 1)])

    pltpu.sync_copy(idx_hbm.at[pl.ds(base, RPW)], idx_v)
    for k in range(CH):
        pltpu.sync_copy(pre_hbm, pre_v.at[pl.ds(k, 1)])
    for k in range(RPW):
        pltpu.sync_copy(tok_hbm, tok_v.at[pl.ds(k, 1)])
    pending = [pltpu.async_copy(tok_v, out_tok.at[pl.ds(base, RPW)], ssem)]

    plsc.subcore_barrier()

    for c in range(NCH):
        r0 = base + c * CH
        pltpu.async_copy(
            prompt_hbm.at[idx_v.at[pl.ds(c * CH, CH)]], core_v, gsem
        ).wait()
        core_wr = pltpu.async_copy(
            core_v, out_emb.at[pl.ds(r0, CH), pl.ds(D, CORE_W)], osem
        )
        pending.append(pltpu.async_copy(
            pre_v, out_emb.at[pl.ds(r0, CH), pl.ds(0, D)], ssem
        ))
        for k in range(CH // SUF_REP):
            pending.append(pltpu.async_copy(
                suf_sh,
                out_emb.at[pl.ds(r0 + k * SUF_REP, SUF_REP),
                           pl.ds(D + CORE_W, SUF_W)],
                ssem
            ))
        core_wr.wait()

    for p in pending:
        p.wait()


def kernel(indices, mini_batch, prompt, embedding_prefix, embedding_suffix,
           tokenized_prompts):
    del mini_batch
    idx = indices.reshape(-1)
    emb2d, tok = _assemble(
        idx,
        prompt.reshape(prompt.shape[0], CORE_W),
        embedding_prefix.reshape(1, D),
        embedding_suffix.reshape(1, SUF_W),
        tokenized_prompts,
    )
    return emb2d.reshape(ROWS, CTX, D), tok


# speedup vs baseline: 1.6257x; 1.6257x over previous
import functools

import jax
import jax.numpy as jnp
from jax import lax
from jax.experimental import pallas as pl
from jax.experimental.pallas import tpu as pltpu
from jax.experimental.pallas import tpu_sc as plsc

PROMPT_LEN = 16
D = 768
SUF = 110
CTX = 1 + PROMPT_LEN + SUF
HEAD = 24
HSUF = HEAD - 1 - PROMPT_LEN
TAIL = 96
END = CTX - HEAD - TAIL
ROWS = 512
NUM_CORES = 2
NUM_SUBCORES = 16
NW = NUM_CORES * NUM_SUBCORES
RPW = ROWS // NW
CH = 8
NCH = RPW // CH
NBLK = 13
ROWBLK = 64

_mesh = plsc.VectorSubcoreMesh(core_axis_name="c", subcore_axis_name="s")


@functools.partial(
    pl.kernel,
    out_type=(
        jax.ShapeDtypeStruct((ROWS, CTX, D), jnp.float32),
        jax.ShapeDtypeStruct((ROWS, CTX), jnp.int32),
    ),
    mesh=_mesh,
    scratch_types=[
        pltpu.VMEM((RPW,), jnp.int32),
        pltpu.VMEM((CH, PROMPT_LEN, D), jnp.float32),
        pltpu.VMEM((1, CH, D), jnp.float32),
        pltpu.VMEM((RPW, CTX), jnp.int32),
        pltpu.VMEM_SHARED((NUM_SUBCORES, HEAD, D), jnp.float32),
        pltpu.VMEM_SHARED((1, TAIL, D), jnp.float32),
        pltpu.SemaphoreType.DMA,
        pltpu.SemaphoreType.DMA,
        pltpu.SemaphoreType.DMA,
        pltpu.SemaphoreType.DMA,
    ],
)
def _assemble(idx_hbm, prompt_hbm, pre_hbm, suf_hbm, tok_hbm,
              out_emb, out_tok,
              idx_v, core_v, bounce_v, tok_v, head_sh, tail_sh,
              gsem, lsem, osem, wsem):
    cid = lax.axis_index("c")
    sid = lax.axis_index("s")
    wid = sid * NUM_CORES + cid
    base = wid * RPW
    head_v = head_sh.at[pl.ds(sid, 1)]

    pltpu.sync_copy(suf_hbm.at[:, pl.ds(0, CH)], bounce_v)
    pltpu.sync_copy(pre_hbm, head_v.at[:, pl.ds(0, 1)])
    for i in range(HSUF):
        pltpu.sync_copy(bounce_v.at[:, pl.ds(i, 1)],
                        head_v.at[:, pl.ds(1 + PROMPT_LEN + i, 1)])

    @pl.when(sid == 0)
    def _():
        pltpu.sync_copy(bounce_v.at[:, pl.ds(HSUF, 1)],
                        tail_sh.at[:, pl.ds(0, 1)])

    for s in range(1, NBLK):
        @pl.when(sid == s)
        def _():
            pltpu.sync_copy(suf_hbm.at[:, pl.ds(CH * s, CH)], bounce_v)
            for q in range(CH):
                p = CH * s - HSUF + q
                if p < TAIL:
                    pltpu.sync_copy(bounce_v.at[:, pl.ds(q, 1)],
                                    tail_sh.at[:, pl.ds(p, 1)])

    pltpu.sync_copy(idx_hbm.at[pl.ds(base, RPW)], idx_v)
    for k in range(RPW):
        pltpu.sync_copy(tok_hbm, tok_v.at[pl.ds(k, 1)])
    pending = [pltpu.async_copy(tok_v, out_tok.at[pl.ds(base, RPW)], wsem)]

    plsc.subcore_barrier()

    for c in range(NCH):
        r0 = base + c * CH
        pltpu.async_copy(
            prompt_hbm.at[idx_v.at[pl.ds(c * CH, CH)]], core_v, gsem
        ).wait()
        for j in range(CH):
            r = r0 + j
            drops = [
                pltpu.async_copy(core_v.at[pl.ds(j, 1), pl.ds(k, 1)],
                                 head_v.at[:, pl.ds(1 + k, 1)], lsem)
                for k in range(PROMPT_LEN)
            ]
            for d in drops:
                d.wait()
            head_wr = pltpu.async_copy(
                head_v, out_emb.at[pl.ds(r, 1), pl.ds(0, HEAD)], osem)
            pending.append(pltpu.async_copy(
                tail_sh, out_emb.at[pl.ds(r, 1), pl.ds(HEAD, TAIL)], wsem))
            head_wr.wait()

    for p in pending:
        p.wait()


def _end_body(emb_any, suf_ref, out_ref):
    del emb_any
    tailv = suf_ref[0, pl.ds(SUF - END, END), :]
    blk = jnp.concatenate([tailv, jnp.zeros((1, D), jnp.float32)], axis=0)
    out_ref[...] = jnp.broadcast_to(blk[None], (ROWBLK, 8, D))


_end_tile = pl.pallas_call(
    _end_body,
    grid=(ROWS // ROWBLK,),
    in_specs=[
        pl.BlockSpec(memory_space=pltpu.MemorySpace.HBM),
        pl.BlockSpec((1, SUF, D), lambda i: (0, 0, 0)),
    ],
    out_specs=pl.BlockSpec((ROWBLK, 8, D), lambda i: (i, (HEAD + TAIL) // 8, 0)),
    out_shape=jax.ShapeDtypeStruct((ROWS, CTX, D), jnp.float32),
    input_output_aliases={0: 0},
)


def kernel(indices, mini_batch, prompt, embedding_prefix, embedding_suffix,
           tokenized_prompts):
    del mini_batch
    emb, tok = _assemble(indices.reshape(-1), prompt, embedding_prefix,
                         embedding_suffix, tokenized_prompts)
    emb = _end_tile(emb, embedding_suffix)
    return emb, tok
